# baseline (speedup 1.0000x reference)
import functools
import jax
import jax.numpy as jnp
from jax import lax
from jax.experimental import pallas as pl
from jax.experimental.pallas import tpu as pltpu
from jax.experimental.pallas import tpu_sc as plsc


def _fused_kernel(x_ref, w_ref, bt_ref, biaT_ref, labT_ref,
                  ap_ref, op_ref, ag_ref, og_ref,
                  m_ref, lS_ref, lE_ref, la_ref, lo_ref,
                  acc_ref):
    b = pl.program_id(0)
    m = pl.program_id(1)
    nb = pl.num_programs(0)
    nm = pl.num_programs(1)
    blk = x_ref.shape[1]

    @pl.when((b == 0) & (m == 0))
    def _init():
        for i in range(6):
            acc_ref[i] = jnp.float32(0.0)

    lt2 = jnp.dot(x_ref[0], w_ref[...], preferred_element_type=jnp.float32)
    ltT = jnp.transpose(lt2)
    xsc = (ltT + bt_ref[...]) * (1.0 + biaT_ref[0])
    labT = labT_ref[0]
    w = (labT[0:1, :] >= 0).astype(jnp.float32)
    y = labT.astype(jnp.float32)
    elem = jnp.maximum(xsc, 0.0) - xsc * y + jnp.log1p(jnp.exp(-jnp.abs(xsc)))
    welem = w * elem
    acc_ref[0] += jnp.sum(welem[0:1, :])
    acc_ref[1] += jnp.sum(welem[1:2, :])
    pred = jax.nn.sigmoid(xsc) * w
    m_ref[0, :, pl.ds(m * blk, blk)] = jax.lax.bitcast_convert_type(
        pred, jnp.int32)

    @pl.when(m == 0)
    def _ce():
        def ce_acc(x, tgt, i0):
            mx = jnp.max(x, axis=0, keepdims=True)
            lse = mx + jnp.log(jnp.sum(jnp.exp(x - mx), axis=0, keepdims=True))
            valid = tgt != -1
            vf = valid.astype(jnp.float32)
            st = jnp.where(valid, tgt, 0)
            oh = (jax.lax.broadcasted_iota(jnp.int32, x.shape, 0) == st
                  ).astype(jnp.float32)
            xt = jnp.sum(x * oh, axis=0, keepdims=True)
            acc_ref[i0] += jnp.sum((lse - xt) * vf)
            acc_ref[i0 + 1] += jnp.sum(vf)

        ce_acc(ap_ref[0], ag_ref[0], 2)
        ce_acc(op_ref[0], og_ref[0], 4)

    @pl.when((b == nb - 1) & (m == nm - 1))
    def _final():
        nelem = jnp.float32(nb * nm * blk)
        lS_ref[...] = jnp.reshape(acc_ref[0] / nelem, (1, 1))
        lE_ref[...] = jnp.reshape(acc_ref[1] / nelem, (1, 1))
        la_ref[...] = jnp.reshape(
            0.1 * acc_ref[2] / jnp.maximum(acc_ref[3], 1.0), (1, 1))
        lo_ref[...] = jnp.reshape(
            0.1 * acc_ref[4] / jnp.maximum(acc_ref[5], 1.0), (1, 1))


def _make_sc_prune(B, LL, L):
    NV = LL // 16
    mesh = plsc.VectorSubcoreMesh(core_axis_name="c", subcore_axis_name="s")

    @functools.partial(
        pl.kernel, mesh=mesh,
        out_type=jax.ShapeDtypeStruct((B, 2, LL), jnp.int32),
        scratch_types=[
            pltpu.VMEM((LL,), jnp.int32),
            pltpu.VMEM((LL,), jnp.int32),
            pltpu.VMEM((L,), jnp.int32),
        ],
        compiler_params=pltpu.CompilerParams(needs_layout_passes=False),
    )
    def sc_prune(bits_hbm, am_hbm, out_hbm, bits_v, mask_v, am_v):
        wid = lax.axis_index("s") * 2 + lax.axis_index("c")

        @pl.when(wid < B * 2)
        def _work():
            b = wid // 2
            ch = wid % 2
            pltpu.sync_copy(bits_hbm.at[b, ch], bits_v)
            pltpu.sync_copy(am_hbm.at[b], am_v)

            def sum_row(i, acc):
                v = am_v[pl.ds(i * 16, 16)]
                return acc + plsc.all_reduce_population_count(v != 0)

            mask_len = lax.fori_loop(0, L // 16, sum_row,
                                     jnp.zeros((16,), jnp.int32)) - 2
            length = (mask_len.astype(jnp.float32) * 0.3).astype(jnp.int32)
            length = jnp.maximum(length, 5)
            length = jnp.minimum(length, mask_len * mask_len)

            def count_ge(t):
                def body(i, acc):
                    v = bits_v[pl.ds(i * 16, 16)]
                    return acc + plsc.all_reduce_population_count(v >= t)

                return lax.fori_loop(0, NV, body, jnp.zeros((16,), jnp.int32))

            def bis(_, lh):
                lo, hi = lh
                mid = lax.shift_right_arithmetic(lo + hi, jnp.int32(1))
                ok = count_ge(mid) >= length
                return (jnp.where(ok, mid, lo), jnp.where(ok, hi, mid))

            lo, _ = lax.fori_loop(
                0, 31, bis,
                (jnp.zeros((16,), jnp.int32),
                 jnp.full((16,), 0x3F800001, jnp.int32)))

            def emit(i, _):
                v = bits_v[pl.ds(i * 16, 16)]
                mask_v[pl.ds(i * 16, 16)] = jnp.where(
                    v >= lo, jnp.int32(1), jnp.int32(0))
                return 0

            lax.fori_loop(0, NV, emit, 0)
            pltpu.sync_copy(mask_v, out_hbm.at[b, ch])

    return sc_prune


def kernel(table, attention_mask, table_labels_S, table_labels_E,
           aspect_pred_tags, opinion_pred_tags, aspect_golde_tags,
           opinion_golde_tags, biaffine_edge_S, biaffine_edge_E,
           W_S, b_S, W_E, b_E):
    B, Lq, Lk, D = table.shape
    LL = Lq * Lk
    BLK = 8192
    NM = LL // BLK

    x = table.reshape(B, LL, D)
    W2 = jnp.concatenate([W_S, W_E], axis=1)
    b2T = jnp.concatenate([b_S, b_E]).reshape(2, 1)
    biaT = jnp.stack([biaffine_edge_S.reshape(B, LL),
                      biaffine_edge_E.reshape(B, LL)], axis=1)
    labT = jnp.stack([table_labels_S.reshape(B, LL),
                      table_labels_E.reshape(B, LL)], axis=1)
    ap_t = jnp.transpose(aspect_pred_tags, (0, 2, 1))
    op_t = jnp.transpose(opinion_pred_tags, (0, 2, 1))
    ag3 = aspect_golde_tags.reshape(B, 1, Lq)
    og3 = opinion_golde_tags.reshape(B, 1, Lq)
    C = ap_t.shape[1]

    outs = pl.pallas_call(
        _fused_kernel,
        grid=(B, NM),
        in_specs=[
            pl.BlockSpec((1, BLK, D), lambda b, m: (b, m, 0)),
            pl.BlockSpec((D, 2), lambda b, m: (0, 0)),
            pl.BlockSpec((2, 1), lambda b, m: (0, 0)),
            pl.BlockSpec((1, 2, BLK), lambda b, m: (b, 0, m)),
            pl.BlockSpec((1, 2, BLK), lambda b, m: (b, 0, m)),
            pl.BlockSpec((1, C, Lq), lambda b, m: (b, 0, 0)),
            pl.BlockSpec((1, C, Lq), lambda b, m: (b, 0, 0)),
            pl.BlockSpec((1, 1, Lq), lambda b, m: (b, 0, 0)),
            pl.BlockSpec((1, 1, Lq), lambda b, m: (b, 0, 0)),
        ],
        out_specs=[
            pl.BlockSpec((1, 2, LL), lambda b, m: (b, 0, 0)),
            pl.BlockSpec((1, 1), lambda b, m: (0, 0)),
            pl.BlockSpec((1, 1), lambda b, m: (0, 0)),
            pl.BlockSpec((1, 1), lambda b, m: (0, 0)),
            pl.BlockSpec((1, 1), lambda b, m: (0, 0)),
        ],
        out_shape=[
            jax.ShapeDtypeStruct((B, 2, LL), jnp.int32),
            jax.ShapeDtypeStruct((1, 1), jnp.float32),
            jax.ShapeDtypeStruct((1, 1), jnp.float32),
            jax.ShapeDtypeStruct((1, 1), jnp.float32),
            jax.ShapeDtypeStruct((1, 1), jnp.float32),
        ],
        scratch_shapes=[
            pltpu.SMEM((8,), jnp.float32),
        ],
    )(x, W2, b2T, biaT, labT, ap_t, op_t, ag3, og3)

    bits, lS, lE, la, lo = outs
    masks = _make_sc_prune(B, LL, Lq)(bits, attention_mask)
    mS = masks[:, 0, :].reshape(B, Lq, Lk).astype(jnp.bool_)
    mE = masks[:, 1, :].reshape(B, Lq, Lk).astype(jnp.bool_)
    return (lS.reshape(()), lE.reshape(()), la.reshape(()), lo.reshape(()),
            mS, mE)


# speedup vs baseline: 1.8407x; 1.8407x over previous
import jax
import jax.numpy as jnp
from jax.experimental import pallas as pl
from jax.experimental.pallas import tpu as pltpu


def _fused_kernel(x_ref, w_ref, bt_ref, biaT_ref, labT_ref, am_ref,
                  ap_ref, op_ref, ag_ref, og_ref,
                  m_ref, lS_ref, lE_ref, la_ref, lo_ref,
                  bits_ref, acc_ref):
    b = pl.program_id(0)
    m = pl.program_id(1)
    nb = pl.num_programs(0)
    nm = pl.num_programs(1)
    blk = x_ref.shape[1]

    @pl.when((b == 0) & (m == 0))
    def _init():
        for i in range(6):
            acc_ref[i] = jnp.float32(0.0)

    lt2 = jnp.dot(x_ref[0], w_ref[...], preferred_element_type=jnp.float32)
    ltT = jnp.transpose(lt2)
    xsc = (ltT + bt_ref[...]) * (1.0 + biaT_ref[0])
    labT = labT_ref[0]
    w = (labT[0:1, :] >= 0).astype(jnp.float32)
    y = labT.astype(jnp.float32)
    elem = jnp.maximum(xsc, 0.0) - xsc * y + jnp.log1p(jnp.exp(-jnp.abs(xsc)))
    welem = w * elem
    acc_ref[0] += jnp.sum(welem[0:1, :])
    acc_ref[1] += jnp.sum(welem[1:2, :])
    pred = jax.nn.sigmoid(xsc) * w
    bits_ref[:, pl.ds(m * blk, blk)] = jax.lax.bitcast_convert_type(
        pred, jnp.int32)

    @pl.when(m == 0)
    def _ce():
        def ce_acc(x, tgt, i0):
            mx = jnp.max(x, axis=0, keepdims=True)
            lse = mx + jnp.log(jnp.sum(jnp.exp(x - mx), axis=0, keepdims=True))
            valid = tgt != -1
            vf = valid.astype(jnp.float32)
            st = jnp.where(valid, tgt, 0)
            oh = (jax.lax.broadcasted_iota(jnp.int32, x.shape, 0) == st
                  ).astype(jnp.float32)
            xt = jnp.sum(x * oh, axis=0, keepdims=True)
            acc_ref[i0] += jnp.sum((lse - xt) * vf)
            acc_ref[i0 + 1] += jnp.sum(vf)

        ce_acc(ap_ref[0], ag_ref[0], 2)
        ce_acc(op_ref[0], og_ref[0], 4)

    @pl.when(m == nm - 1)
    def _prune():
        mask_len = jnp.sum(am_ref[0]) - 2
        length = (mask_len.astype(jnp.float32) * 0.3).astype(jnp.int32)
        length = jnp.maximum(length, 5)
        length = jnp.minimum(length, mask_len * mask_len)
        bits = bits_ref[...]

        def body(_, lh):
            lo, hi = lh
            mid = jax.lax.div(lo + hi, jnp.int32(2))
            cnt = jnp.sum((bits >= mid).astype(jnp.int32), axis=1,
                          keepdims=True)
            ok = cnt >= length
            return jnp.where(ok, mid, lo), jnp.where(ok, hi, mid)

        lo, _ = jax.lax.fori_loop(
            0, 31, body,
            (jnp.zeros((2, 1), jnp.int32),
             jnp.full((2, 1), 0x3F800001, jnp.int32)))
        m_ref[0] = (bits >= lo).astype(jnp.int32)

    @pl.when((b == nb - 1) & (m == nm - 1))
    def _final():
        nelem = jnp.float32(nb * nm * blk)
        lS_ref[...] = jnp.reshape(acc_ref[0] / nelem, (1, 1))
        lE_ref[...] = jnp.reshape(acc_ref[1] / nelem, (1, 1))
        la_ref[...] = jnp.reshape(
            0.1 * acc_ref[2] / jnp.maximum(acc_ref[3], 1.0), (1, 1))
        lo_ref[...] = jnp.reshape(
            0.1 * acc_ref[4] / jnp.maximum(acc_ref[5], 1.0), (1, 1))


def kernel(table, attention_mask, table_labels_S, table_labels_E,
           aspect_pred_tags, opinion_pred_tags, aspect_golde_tags,
           opinion_golde_tags, biaffine_edge_S, biaffine_edge_E,
           W_S, b_S, W_E, b_E):
    B, Lq, Lk, D = table.shape
    LL = Lq * Lk
    BLK = 8192
    NM = LL // BLK

    x = table.reshape(B, LL, D)
    W2 = jnp.concatenate([W_S, W_E], axis=1)
    b2T = jnp.concatenate([b_S, b_E]).reshape(2, 1)
    biaT = jnp.stack([biaffine_edge_S.reshape(B, LL),
                      biaffine_edge_E.reshape(B, LL)], axis=1)
    labT = jnp.stack([table_labels_S.reshape(B, LL),
                      table_labels_E.reshape(B, LL)], axis=1)
    am3 = attention_mask.reshape(B, 1, Lq)
    ap_t = jnp.transpose(aspect_pred_tags, (0, 2, 1))
    op_t = jnp.transpose(opinion_pred_tags, (0, 2, 1))
    ag3 = aspect_golde_tags.reshape(B, 1, Lq)
    og3 = opinion_golde_tags.reshape(B, 1, Lq)
    C = ap_t.shape[1]

    outs = pl.pallas_call(
        _fused_kernel,
        grid=(B, NM),
        in_specs=[
            pl.BlockSpec((1, BLK, D), lambda b, m: (b, m, 0)),
            pl.BlockSpec((D, 2), lambda b, m: (0, 0)),
            pl.BlockSpec((2, 1), lambda b, m: (0, 0)),
            pl.BlockSpec((1, 2, BLK), lambda b, m: (b, 0, m)),
            pl.BlockSpec((1, 2, BLK), lambda b, m: (b, 0, m)),
            pl.BlockSpec((1, 1, Lq), lambda b, m: (b, 0, 0)),
            pl.BlockSpec((1, C, Lq), lambda b, m: (b, 0, 0)),
            pl.BlockSpec((1, C, Lq), lambda b, m: (b, 0, 0)),
            pl.BlockSpec((1, 1, Lq), lambda b, m: (b, 0, 0)),
            pl.BlockSpec((1, 1, Lq), lambda b, m: (b, 0, 0)),
        ],
        out_specs=[
            pl.BlockSpec((1, 2, LL), lambda b, m: (b, 0, 0)),
            pl.BlockSpec((1, 1), lambda b, m: (0, 0)),
            pl.BlockSpec((1, 1), lambda b, m: (0, 0)),
            pl.BlockSpec((1, 1), lambda b, m: (0, 0)),
            pl.BlockSpec((1, 1), lambda b, m: (0, 0)),
        ],
        out_shape=[
            jax.ShapeDtypeStruct((B, 2, LL), jnp.int32),
            jax.ShapeDtypeStruct((1, 1), jnp.float32),
            jax.ShapeDtypeStruct((1, 1), jnp.float32),
            jax.ShapeDtypeStruct((1, 1), jnp.float32),
            jax.ShapeDtypeStruct((1, 1), jnp.float32),
        ],
        scratch_shapes=[
            pltpu.VMEM((2, LL), jnp.int32),
            pltpu.SMEM((8,), jnp.float32),
        ],
    )(x, W2, b2T, biaT, labT, am3, ap_t, op_t, ag3, og3)

    masks, lS, lE, la, lo = outs
    mS = masks[:, 0, :].reshape(B, Lq, Lk).astype(jnp.bool_)
    mE = masks[:, 1, :].reshape(B, Lq, Lk).astype(jnp.bool_)
    return (lS.reshape(()), lE.reshape(()), la.reshape(()), lo.reshape(()),
            mS, mE)
